# Initial kernel scaffold; baseline (speedup 1.0000x reference)
import jax
import jax.numpy as jnp
from jax.experimental import pallas as pl


def kernel(x, tables):
    raise NotImplementedError("write your pallas kernel here")


# speedup vs baseline: 1.0461x; 1.0461x over previous
import functools

import jax
import jax.numpy as jnp
from jax import lax
from jax.experimental import pallas as pl
from jax.experimental.pallas import tpu as pltpu
from jax.experimental.pallas import tpu_sc as plsc

_F = 26
_V = 100000
_D = 64
_B = 16384
_NW = 32
_BPW = _B // _NW
_CH = 128
_NCH = _BPW // _CH
_LANES = 16

_mesh = plsc.VectorSubcoreMesh(core_axis_name="c", subcore_axis_name="s")


@functools.partial(
    pl.kernel,
    mesh=_mesh,
    compiler_params=pltpu.CompilerParams(use_tc_tiling_on_sc=False),
    out_type=jax.ShapeDtypeStruct((_B, _D), jnp.float32),
    scratch_types=[
        pltpu.VMEM((_F, _BPW), jnp.int32),
        pltpu.VMEM((_BPW, _D), jnp.float32),
        pltpu.VMEM((_BPW, _D), jnp.float32),
        pltpu.VMEM((_BPW, _D), jnp.float32),
        pltpu.SemaphoreType.DMA,
        pltpu.SemaphoreType.DMA,
        pltpu.SemaphoreType.DMA,
    ],
)
def _encode(tf_hbm, xt_hbm, out_hbm, idx_v, buf_a, buf_b, acc_v,
            sem_a, sem_b, sem_acc):
    wid = lax.axis_index("s") * 2 + lax.axis_index("c")
    base = wid * _BPW

    pltpu.sync_copy(xt_hbm.at[:, pl.ds(base, _BPW)], idx_v)

    def _fire(i, dst, sem):
        return [
            pltpu.async_copy(
                tf_hbm.at[idx_v.at[i, pl.ds(j * _CH, _CH)]],
                dst.at[pl.ds(j * _CH, _CH)],
                sem,
            )
            for j in range(_NCH)
        ]

    h0 = _fire(0, acc_v, sem_acc)

    for i in range(1, _F):
        def _ofs(k, _, i=i):
            sl = pl.ds(k * _LANES, _LANES)
            idx_v[i, sl] = idx_v[i, sl] + (i * _V)
            return 0
        lax.fori_loop(0, _BPW // _LANES, _ofs, 0)

    def _accum(src):
        def _body(r, _):
            for c in range(_D // _LANES):
                sl = pl.ds(c * _LANES, _LANES)
                plsc.addupdate(acc_v.at[r, sl], src[r, sl])
            return 0
        lax.fori_loop(0, _BPW, _body, 0)

    bufs = [buf_a, buf_b]
    sems = [sem_a, sem_b]
    pending = _fire(1, bufs[1], sems[1])
    for d in h0:
        d.wait()
    for i in range(2, _F):
        nxt = _fire(i, bufs[i % 2], sems[i % 2])
        for d in pending:
            d.wait()
        _accum(bufs[(i - 1) % 2])
        pending = nxt
    for d in pending:
        d.wait()
    _accum(bufs[(_F - 1) % 2])

    pltpu.sync_copy(acc_v, out_hbm.at[pl.ds(base, _BPW)])


def kernel(x, tables):
    xt = x.astype(jnp.int32).T
    tf = tables.reshape(_F * _V, _D)
    return _encode(tf, xt)
